# baseline (speedup 1.0000x reference)
import functools

import jax
import jax.numpy as jnp
from jax.experimental import pallas as pl
from jax.experimental.pallas import tpu as pltpu

_BK = 4096
_BC = 512
_BIGF = 3e7


def _nn_body(q_ref, k_ref, top_sim_ref, top_idx_ref, avg_ref,
             qn_ref, idxf_ref, ksum_ref, kn0_ref, *, nblk, bk, k_total):
    i = pl.program_id(0)
    npad = nblk * bk - k_total

    @pl.when(i == 0)
    def _init():
        q = q_ref[...]
        qn_ref[...] = q / (jnp.sqrt(jnp.sum(q * q, axis=1, keepdims=True)) + 1e-8)
        top_sim_ref[...] = jnp.full(top_sim_ref.shape, -jnp.inf, jnp.float32)
        idxf_ref[...] = jnp.zeros(idxf_ref.shape, jnp.float32)
        ksum_ref[...] = jnp.zeros(ksum_ref.shape, jnp.float32)

    nch = bk // _BC
    for c in range(nch):
        kb = k_ref[c * _BC:(c + 1) * _BC, :]
        kn = kb / (jnp.sqrt(jnp.sum(kb * kb, axis=1, keepdims=True)) + 1e-8)
        ksum_ref[...] += jnp.sum(kn, axis=0, keepdims=True)

        if c == 0:
            @pl.when(i == 0)
            def _save_kn0():
                kn0_ref[...] = kn[0:1, :]

        sim = jax.lax.dot_general(
            qn_ref[...], kn, (((1,), (1,)), ((), ())),
            preferred_element_type=jnp.float32)

        local_max = jnp.max(sim, axis=1, keepdims=True)
        colf = jax.lax.broadcasted_iota(
            jnp.int32, sim.shape, 1).astype(jnp.float32)
        cand = jnp.where(sim == local_max, colf, _BIGF)
        local_argf = jnp.min(cand, axis=1, keepdims=True)
        better = local_max > top_sim_ref[...]
        idxf_ref[...] = jnp.where(
            better, local_argf + jnp.float32(i * bk + c * _BC),
            idxf_ref[...])
        top_sim_ref[...] = jnp.where(better, local_max, top_sim_ref[...])

    @pl.when(i == nblk - 1)
    def _finalize():
        ks = ksum_ref[...] - jnp.float32(npad) * kn0_ref[...]
        avg = jax.lax.dot_general(
            qn_ref[...], ks, (((1,), (1,)), ((), ())),
            preferred_element_type=jnp.float32)
        avg_ref[...] = avg * (1.0 / k_total)
        top_idx_ref[...] = idxf_ref[...].astype(jnp.int32)


def kernel(queries, keys, db_classes):
    q, d = queries.shape
    k = keys.shape[0]
    k_pad = ((k + _BK - 1) // _BK) * _BK
    nblk = k_pad // _BK
    if k_pad != k:
        keys = jnp.concatenate(
            [keys, jnp.broadcast_to(keys[0:1, :], (k_pad - k, keys.shape[1]))],
            axis=0)

    top_sim, top_idx, avg_sim = pl.pallas_call(
        functools.partial(_nn_body, nblk=nblk, bk=_BK, k_total=k),
        grid=(nblk,),
        in_specs=[
            pl.BlockSpec((q, d), lambda i: (0, 0)),
            pl.BlockSpec((_BK, d), lambda i: (i, 0)),
        ],
        out_specs=[
            pl.BlockSpec((q, 1), lambda i: (0, 0)),
            pl.BlockSpec((q, 1), lambda i: (0, 0)),
            pl.BlockSpec((q, 1), lambda i: (0, 0)),
        ],
        out_shape=[
            jax.ShapeDtypeStruct((q, 1), jnp.float32),
            jax.ShapeDtypeStruct((q, 1), jnp.int32),
            jax.ShapeDtypeStruct((q, 1), jnp.float32),
        ],
        scratch_shapes=[
            pltpu.VMEM((q, d), jnp.float32),
            pltpu.VMEM((q, 1), jnp.float32),
            pltpu.VMEM((1, d), jnp.float32),
            pltpu.VMEM((1, d), jnp.float32),
        ],
        compiler_params=pltpu.CompilerParams(
            dimension_semantics=("arbitrary",)),
    )(queries, keys)

    top_cls = jnp.take(db_classes, top_idx[:, 0], axis=0)
    return (top_sim, top_cls, avg_sim[:, 0])


# speedup vs baseline: 1.1951x; 1.1951x over previous
import functools

import jax
import jax.numpy as jnp
from jax.experimental import pallas as pl
from jax.experimental.pallas import tpu as pltpu

_BK = 8192
_BC = 1024
_BIGF = 3e7


def _nn_body(q_ref, k_ref, top_sim_ref, top_idx_ref, avg_ref,
             qn_ref, idxf_ref, ksum_ref, kn0_ref, *, nblk, bk, k_total):
    i = pl.program_id(0)
    npad = nblk * bk - k_total

    @pl.when(i == 0)
    def _init():
        q = q_ref[...]
        qn_ref[...] = q / (jnp.sqrt(jnp.sum(q * q, axis=1, keepdims=True)) + 1e-8)
        top_sim_ref[...] = jnp.full(top_sim_ref.shape, -jnp.inf, jnp.float32)
        idxf_ref[...] = jnp.zeros(idxf_ref.shape, jnp.float32)
        ksum_ref[...] = jnp.zeros(ksum_ref.shape, jnp.float32)

    nch = bk // _BC
    for c in range(nch):
        kb = k_ref[c * _BC:(c + 1) * _BC, :]
        kn = kb / (jnp.sqrt(jnp.sum(kb * kb, axis=1, keepdims=True)) + 1e-8)
        ksum_ref[...] += jnp.sum(kn, axis=0, keepdims=True)

        if c == 0:
            @pl.when(i == 0)
            def _save_kn0():
                kn0_ref[...] = kn[0:1, :]

        sim = jax.lax.dot_general(
            qn_ref[...], kn, (((1,), (1,)), ((), ())),
            preferred_element_type=jnp.float32)

        local_max = jnp.max(sim, axis=1, keepdims=True)
        colf = jax.lax.broadcasted_iota(
            jnp.int32, sim.shape, 1).astype(jnp.float32)
        cand = jnp.where(sim == local_max, colf, _BIGF)
        local_argf = jnp.min(cand, axis=1, keepdims=True)
        better = local_max > top_sim_ref[...]
        idxf_ref[...] = jnp.where(
            better, local_argf + jnp.float32(i * bk + c * _BC),
            idxf_ref[...])
        top_sim_ref[...] = jnp.where(better, local_max, top_sim_ref[...])

    @pl.when(i == nblk - 1)
    def _finalize():
        ks = ksum_ref[...] - jnp.float32(npad) * kn0_ref[...]
        avg = jax.lax.dot_general(
            qn_ref[...], ks, (((1,), (1,)), ((), ())),
            preferred_element_type=jnp.float32)
        avg_ref[...] = avg * (1.0 / k_total)
        top_idx_ref[...] = idxf_ref[...].astype(jnp.int32)


def kernel(queries, keys, db_classes):
    q, d = queries.shape
    k = keys.shape[0]
    k_pad = ((k + _BK - 1) // _BK) * _BK
    nblk = k_pad // _BK
    if k_pad != k:
        keys = jnp.concatenate(
            [keys, jnp.broadcast_to(keys[0:1, :], (k_pad - k, keys.shape[1]))],
            axis=0)

    top_sim, top_idx, avg_sim = pl.pallas_call(
        functools.partial(_nn_body, nblk=nblk, bk=_BK, k_total=k),
        grid=(nblk,),
        in_specs=[
            pl.BlockSpec((q, d), lambda i: (0, 0)),
            pl.BlockSpec((_BK, d), lambda i: (i, 0)),
        ],
        out_specs=[
            pl.BlockSpec((q, 1), lambda i: (0, 0)),
            pl.BlockSpec((q, 1), lambda i: (0, 0)),
            pl.BlockSpec((q, 1), lambda i: (0, 0)),
        ],
        out_shape=[
            jax.ShapeDtypeStruct((q, 1), jnp.float32),
            jax.ShapeDtypeStruct((q, 1), jnp.int32),
            jax.ShapeDtypeStruct((q, 1), jnp.float32),
        ],
        scratch_shapes=[
            pltpu.VMEM((q, d), jnp.float32),
            pltpu.VMEM((q, 1), jnp.float32),
            pltpu.VMEM((1, d), jnp.float32),
            pltpu.VMEM((1, d), jnp.float32),
        ],
        compiler_params=pltpu.CompilerParams(
            dimension_semantics=("arbitrary",)),
    )(queries, keys)

    top_cls = jnp.take(db_classes, top_idx[:, 0], axis=0)
    return (top_sim, top_cls, avg_sim[:, 0])


# speedup vs baseline: 1.2157x; 1.0173x over previous
import functools

import jax
import jax.numpy as jnp
from jax.experimental import pallas as pl
from jax.experimental.pallas import tpu as pltpu

_BK = 4096
_BC = 1024
_BIGF = 3e7


def _nn_body(q_ref, k_ref, top_sim_ref, top_idx_ref, avg_ref,
             qn_ref, idxf_ref, ksum_ref, kn0_ref, *, nblk, bk, k_total):
    i = pl.program_id(0)
    npad = nblk * bk - k_total

    @pl.when(i == 0)
    def _init():
        q = q_ref[...]
        qn_ref[...] = q / (jnp.sqrt(jnp.sum(q * q, axis=1, keepdims=True)) + 1e-8)
        top_sim_ref[...] = jnp.full(top_sim_ref.shape, -jnp.inf, jnp.float32)
        idxf_ref[...] = jnp.zeros(idxf_ref.shape, jnp.float32)
        ksum_ref[...] = jnp.zeros(ksum_ref.shape, jnp.float32)

    nch = bk // _BC
    for c in range(nch):
        kb = k_ref[c * _BC:(c + 1) * _BC, :]
        kn = kb / (jnp.sqrt(jnp.sum(kb * kb, axis=1, keepdims=True)) + 1e-8)
        ksum_ref[...] += jnp.sum(kn, axis=0, keepdims=True)

        if c == 0:
            @pl.when(i == 0)
            def _save_kn0():
                kn0_ref[...] = kn[0:1, :]

        sim = jax.lax.dot_general(
            qn_ref[...], kn, (((1,), (1,)), ((), ())),
            preferred_element_type=jnp.float32)

        local_max = jnp.max(sim, axis=1, keepdims=True)
        lanef = jax.lax.broadcasted_iota(
            jnp.int32, (sim.shape[0], 128), 1).astype(jnp.float32)
        cmins = [
            jnp.where(sim[:, s * 128:(s + 1) * 128] == local_max,
                      lanef + jnp.float32(s * 128), _BIGF)
            for s in range(_BC // 128)
        ]
        while len(cmins) > 1:
            cmins = [jnp.minimum(a, b) for a, b in zip(cmins[::2], cmins[1::2])]
        local_argf = jnp.min(cmins[0], axis=1, keepdims=True)
        better = local_max > top_sim_ref[...]
        idxf_ref[...] = jnp.where(
            better, local_argf + jnp.float32(i * bk + c * _BC),
            idxf_ref[...])
        top_sim_ref[...] = jnp.where(better, local_max, top_sim_ref[...])

    @pl.when(i == nblk - 1)
    def _finalize():
        ks = ksum_ref[...] - jnp.float32(npad) * kn0_ref[...]
        avg = jax.lax.dot_general(
            qn_ref[...], ks, (((1,), (1,)), ((), ())),
            preferred_element_type=jnp.float32)
        avg_ref[...] = avg * (1.0 / k_total)
        top_idx_ref[...] = idxf_ref[...].astype(jnp.int32)


def kernel(queries, keys, db_classes):
    q, d = queries.shape
    k = keys.shape[0]
    k_pad = ((k + _BK - 1) // _BK) * _BK
    nblk = k_pad // _BK
    if k_pad != k:
        keys = jnp.concatenate(
            [keys, jnp.broadcast_to(keys[0:1, :], (k_pad - k, keys.shape[1]))],
            axis=0)

    top_sim, top_idx, avg_sim = pl.pallas_call(
        functools.partial(_nn_body, nblk=nblk, bk=_BK, k_total=k),
        grid=(nblk,),
        in_specs=[
            pl.BlockSpec((q, d), lambda i: (0, 0)),
            pl.BlockSpec((_BK, d), lambda i: (i, 0)),
        ],
        out_specs=[
            pl.BlockSpec((q, 1), lambda i: (0, 0)),
            pl.BlockSpec((q, 1), lambda i: (0, 0)),
            pl.BlockSpec((q, 1), lambda i: (0, 0)),
        ],
        out_shape=[
            jax.ShapeDtypeStruct((q, 1), jnp.float32),
            jax.ShapeDtypeStruct((q, 1), jnp.int32),
            jax.ShapeDtypeStruct((q, 1), jnp.float32),
        ],
        scratch_shapes=[
            pltpu.VMEM((q, d), jnp.float32),
            pltpu.VMEM((q, 1), jnp.float32),
            pltpu.VMEM((1, d), jnp.float32),
            pltpu.VMEM((1, d), jnp.float32),
        ],
        compiler_params=pltpu.CompilerParams(
            dimension_semantics=("arbitrary",)),
    )(queries, keys)

    top_cls = jnp.take(db_classes, top_idx[:, 0], axis=0)
    return (top_sim, top_cls, avg_sim[:, 0])
